# baseline (speedup 1.0000x reference)
import functools

import jax
import jax.numpy as jnp
from jax import lax
from jax.experimental import pallas as pl
from jax.experimental.pallas import tpu as pltpu
from jax.experimental.pallas import tpu_sc as plsc

_N_FIELDS = 26
_VOCAB = 100000
_EDIM = 32
_BATCH = 16384
_NW = 32
_NPLANES = _N_FIELDS * _EDIM
_PLANES_PER_W = _NPLANES // _NW
_LANES = 16
_UNROLL = 16
_BCHUNK = 2048
_NB = _BATCH // _BCHUNK

_mesh = plsc.VectorSubcoreMesh(core_axis_name="c", subcore_axis_name="s")


@functools.partial(
    pl.kernel,
    mesh=_mesh,
    out_type=jax.ShapeDtypeStruct((_NPLANES, _BATCH), jnp.float32),
    compiler_params=pltpu.CompilerParams(
        use_tc_tiling_on_sc=True, needs_layout_passes=False
    ),
    scratch_types=[
        pltpu.VMEM((_VOCAB,), jnp.float32),
        pltpu.VMEM((_BCHUNK,), jnp.int32),
        pltpu.VMEM((_BCHUNK,), jnp.int32),
        pltpu.VMEM((_BCHUNK,), jnp.float32),
        pltpu.VMEM((_BCHUNK,), jnp.float32),
        pltpu.SemaphoreType.DMA,
        pltpu.SemaphoreType.DMA,
        pltpu.SemaphoreType.DMA,
        pltpu.SemaphoreType.DMA,
    ],
)
def _mk_gather(
    xt_hbm, tt_hbm, out_hbm, plane_v, xa, xb, ra, rb, sxa, sxb, sra, srb
):
    wid = lax.axis_index("s") * 2 + lax.axis_index("c")
    xbufs, xsems = (xa, xb), (sxa, sxb)
    rbufs, rsems = (ra, rb), (sra, srb)

    def do_plane(j, carry):
        c = wid * _PLANES_PER_W + j
        f = c // _EDIM
        e = lax.rem(c, _EDIM)
        hx = pltpu.async_copy(xt_hbm.at[f, pl.ds(0, _BCHUNK)], xbufs[0], xsems[0])
        pltpu.sync_copy(tt_hbm.at[f, e], plane_v)

        row_handles = [None, None]
        for b in range(_NB):
            hx.wait()
            if b + 1 < _NB:
                hx = pltpu.async_copy(
                    xt_hbm.at[f, pl.ds((b + 1) * _BCHUNK, _BCHUNK)],
                    xbufs[(b + 1) % 2],
                    xsems[(b + 1) % 2],
                )
            if row_handles[b % 2] is not None:
                row_handles[b % 2].wait()
            xv = xbufs[b % 2]
            row_v = rbufs[b % 2]

            def gather_group(i, carry3, xv=xv, row_v=row_v):
                base = i * (_LANES * _UNROLL)
                for k in range(_UNROLL):
                    o = base + k * _LANES
                    idx = xv[pl.ds(o, _LANES)]
                    row_v[pl.ds(o, _LANES)] = plsc.load_gather(plane_v, [idx])
                return carry3

            lax.fori_loop(0, _BCHUNK // (_LANES * _UNROLL), gather_group, 0)
            row_handles[b % 2] = pltpu.async_copy(
                row_v,
                out_hbm.at[c, pl.ds(b * _BCHUNK, _BCHUNK)],
                rsems[b % 2],
            )
        row_handles[0].wait()
        row_handles[1].wait()
        return carry

    lax.fori_loop(0, _PLANES_PER_W, do_plane, 0)


def kernel(x, tables):
    xt = x.T
    tt = tables.transpose(0, 2, 1)
    out = _mk_gather(xt, tt)
    return out.T


# speedup vs baseline: 1.1142x; 1.1142x over previous
import functools

import jax
import jax.numpy as jnp
from jax import lax
from jax.experimental import pallas as pl
from jax.experimental.pallas import tpu as pltpu
from jax.experimental.pallas import tpu_sc as plsc

_N_FIELDS = 26
_VOCAB = 100000
_EDIM = 32
_BATCH = 16384
_NW = 32
_NPLANES = _N_FIELDS * _EDIM
_PLANES_PER_W = _NPLANES // _NW
_LANES = 16
_UNROLL = 16
_BCHUNK = 4096
_NB = _BATCH // _BCHUNK

_mesh = plsc.VectorSubcoreMesh(core_axis_name="c", subcore_axis_name="s")


@functools.partial(
    pl.kernel,
    mesh=_mesh,
    out_type=jax.ShapeDtypeStruct((_NPLANES, _BATCH), jnp.float32),
    compiler_params=pltpu.CompilerParams(
        use_tc_tiling_on_sc=True, needs_layout_passes=False
    ),
    scratch_types=[
        pltpu.VMEM((_VOCAB,), jnp.float32),
        pltpu.VMEM((_BCHUNK,), jnp.int32),
        pltpu.VMEM((_BCHUNK,), jnp.int32),
        pltpu.VMEM((_BCHUNK,), jnp.float32),
        pltpu.VMEM((_BCHUNK,), jnp.float32),
        pltpu.SemaphoreType.DMA,
        pltpu.SemaphoreType.DMA,
        pltpu.SemaphoreType.DMA,
        pltpu.SemaphoreType.DMA,
    ],
)
def _mk_gather(
    xt_hbm, tt_hbm, out_hbm, plane_v, xa, xb, ra, rb, sxa, sxb, sra, srb
):
    wid = lax.axis_index("s") * 2 + lax.axis_index("c")
    xbufs, xsems = (xa, xb), (sxa, sxb)
    rbufs, rsems = (ra, rb), (sra, srb)

    def do_plane(j, carry):
        c = wid * _PLANES_PER_W + j
        f = c // _EDIM
        e = lax.rem(c, _EDIM)
        hx = pltpu.async_copy(xt_hbm.at[f, pl.ds(0, _BCHUNK)], xbufs[0], xsems[0])
        pltpu.sync_copy(tt_hbm.at[f, e], plane_v)

        row_handles = [None, None]
        for b in range(_NB):
            hx.wait()
            if b + 1 < _NB:
                hx = pltpu.async_copy(
                    xt_hbm.at[f, pl.ds((b + 1) * _BCHUNK, _BCHUNK)],
                    xbufs[(b + 1) % 2],
                    xsems[(b + 1) % 2],
                )
            if row_handles[b % 2] is not None:
                row_handles[b % 2].wait()
            xv = xbufs[b % 2]
            row_v = rbufs[b % 2]

            def gather_group(i, carry3, xv=xv, row_v=row_v):
                base = i * (_LANES * _UNROLL)
                for k in range(_UNROLL):
                    o = base + k * _LANES
                    idx = xv[pl.ds(o, _LANES)]
                    row_v[pl.ds(o, _LANES)] = plsc.load_gather(plane_v, [idx])
                return carry3

            lax.fori_loop(0, _BCHUNK // (_LANES * _UNROLL), gather_group, 0)
            row_handles[b % 2] = pltpu.async_copy(
                row_v,
                out_hbm.at[c, pl.ds(b * _BCHUNK, _BCHUNK)],
                rsems[b % 2],
            )
        row_handles[0].wait()
        row_handles[1].wait()
        return carry

    lax.fori_loop(0, _PLANES_PER_W, do_plane, 0)


def kernel(x, tables):
    xt = x.T
    tt = tables.transpose(0, 2, 1)
    out = _mk_gather(xt, tt)
    return out.T
